# baseline (speedup 1.0000x reference)
import functools

import jax
import jax.numpy as jnp
from jax import lax
from jax.experimental import pallas as pl
from jax.experimental.pallas import tpu as pltpu
from jax.experimental.pallas import tpu_sc as plsc

NUM_EXPERTS = 8
D_MODEL = 1024
D_FF = 4096
T = 2048

S_TOT = NUM_EXPERTS * T
Y_ROWS = (NUM_EXPERTS + 1) * T
HALF = D_MODEL // 2
FF_CHUNK = 512
N_FF = D_FF // FF_CHUNK
ROW_SUB = 256

_MESH = dict(core_axis_name="c", subcore_axis_name="s")


def _route_body(maskT, tok, cnt, pos, mask_v, tok_v, pos_v, cnt_v):
    c = lax.axis_index("c")
    s = lax.axis_index("s")
    w = s * 2 + c

    @pl.when(w < NUM_EXPERTS)
    def _():
        pltpu.sync_copy(maskT.at[w], mask_v)
        for k in range(len(tok_v) // 16):
            tok_v[pl.ds(k * 16, 16)] = jnp.zeros((16,), jnp.int32)
        lane = lax.broadcasted_iota(jnp.int32, (16,), 0)
        group_base = w * T

        def chunk(k, ptr):
            mv = mask_v[pl.ds(k * 16, 16)]
            ids = lane + k * 16
            exc = plsc.cumsum(mv) - mv
            dst = jnp.where(mv > 0, ptr + exc, T + lane)
            plsc.store_scatter(tok_v, [dst], ids)
            pos_v[pl.ds(k * 16, 16)] = jnp.where(
                mv > 0, group_base + ptr + exc, S_TOT + ids)
            return ptr + jnp.sum(mv)

        ptr = lax.fori_loop(0, T // 16, chunk, jnp.int32(0))
        pltpu.sync_copy(tok_v.at[pl.ds(0, T)], tok.at[w])
        pltpu.sync_copy(pos_v, pos.at[w])
        cnt_v[...] = jnp.full((16,), ptr, jnp.int32)
        pltpu.sync_copy(cnt_v, cnt.at[w])


def _route(maskT):
    return pl.kernel(
        _route_body,
        out_type=(
            jax.ShapeDtypeStruct((NUM_EXPERTS, T), jnp.int32),
            jax.ShapeDtypeStruct((NUM_EXPERTS, 16), jnp.int32),
            jax.ShapeDtypeStruct((NUM_EXPERTS, T), jnp.int32),
        ),
        mesh=plsc.VectorSubcoreMesh(**_MESH),
        scratch_types=(
            pltpu.VMEM((T,), jnp.int32),
            pltpu.VMEM((T + 16, ), jnp.int32),
            pltpu.VMEM((T,), jnp.int32),
            pltpu.VMEM((16,), jnp.int32),
        ),
        compiler_params=pltpu.CompilerParams(needs_layout_passes=False),
    )(maskT)


_G_CHUNK = 64


def _gather_body(x, tok, xg, idx_v, xbuf, sem):
    c = lax.axis_index("c")
    s = lax.axis_index("s")
    w = s * 2 + c
    e = w // 4
    lo = (w % 4) * 512
    for k in range(512 // _G_CHUNK):
        pltpu.sync_copy(tok.at[e, pl.ds(lo + k * _G_CHUNK, _G_CHUNK)],
                        idx_v.at[k])
        pltpu.async_copy(x.at[idx_v.at[k]], xbuf, sem).wait()
        pltpu.sync_copy(xbuf, xg.at[pl.ds(w * 512 + k * _G_CHUNK, _G_CHUNK)])


def _gather(x, tok):
    return pl.kernel(
        _gather_body,
        out_type=jax.ShapeDtypeStruct((S_TOT, D_MODEL), jnp.float32),
        mesh=plsc.VectorSubcoreMesh(**_MESH),
        scratch_types=(
            pltpu.VMEM((512 // _G_CHUNK, _G_CHUNK), jnp.int32),
            pltpu.VMEM((_G_CHUNK, D_MODEL), jnp.float32),
            pltpu.SemaphoreType.DMA,
        ),
    )(x, tok)


def _gmm_body(cnt_ref, xg_ref, wg_ref, wu_ref, wd_ref, yl_ref, yr_ref):
    e = pl.program_id(0)
    f = pl.program_id(1)
    cnt = cnt_ref[e]
    nblk = (cnt + ROW_SUB - 1) // ROW_SUB

    @pl.when((e == NUM_EXPERTS) & (f == 0))
    def _():
        yl_ref[...] = jnp.zeros_like(yl_ref)
        yr_ref[...] = jnp.zeros_like(yr_ref)

    def sub(j, _):
        rows = pl.ds(j * ROW_SUB, ROW_SUB)
        xs = xg_ref[0, rows, :]
        g = jnp.dot(xs, wg_ref[0], preferred_element_type=jnp.float32)
        u = jnp.dot(xs, wu_ref[0], preferred_element_type=jnp.float32)
        h = u * (g * jax.nn.sigmoid(g))
        part = jnp.dot(h, wd_ref[0], preferred_element_type=jnp.float32)

        @pl.when(f == 0)
        def _():
            yl_ref[0, rows, :] = part[:, :HALF]
            yr_ref[0, rows, :] = part[:, HALF:]

        @pl.when(f != 0)
        def _():
            yl_ref[0, rows, :] += part[:, :HALF]
            yr_ref[0, rows, :] += part[:, HALF:]

        return 0

    lax.fori_loop(0, nblk, sub, 0)


def _gmm(counts, xg, gate_up_w, down_w):
    ce = lambda e: jnp.minimum(e, NUM_EXPERTS - 1)
    grid_spec = pltpu.PrefetchScalarGridSpec(
        num_scalar_prefetch=1,
        grid=(NUM_EXPERTS + 1, N_FF),
        in_specs=[
            pl.BlockSpec((1, T, D_MODEL), lambda e, f, cnt: (ce(e), 0, 0)),
            pl.BlockSpec((1, D_MODEL, FF_CHUNK),
                         lambda e, f, cnt: (ce(e), 0, f)),
            pl.BlockSpec((1, D_MODEL, FF_CHUNK),
                         lambda e, f, cnt: (ce(e), 0, N_FF + f)),
            pl.BlockSpec((1, FF_CHUNK, D_MODEL),
                         lambda e, f, cnt: (ce(e), f, 0)),
        ],
        out_specs=[
            pl.BlockSpec((1, T, HALF), lambda e, f, cnt: (e, 0, 0)),
            pl.BlockSpec((1, T, HALF), lambda e, f, cnt: (e, 0, 0)),
        ],
    )
    return pl.pallas_call(
        _gmm_body,
        grid_spec=grid_spec,
        out_shape=(
            jax.ShapeDtypeStruct((NUM_EXPERTS + 1, T, HALF), jnp.float32),
            jax.ShapeDtypeStruct((NUM_EXPERTS + 1, T, HALF), jnp.float32),
        ),
    )(counts, xg.reshape(NUM_EXPERTS, T, D_MODEL), gate_up_w, gate_up_w,
      down_w)


_TOK_PER_TILE = T // 16
_C_SUB = 16


def _combine_body(yl, yr, pos, out, pos_v, gbuf, sem):
    c = lax.axis_index("c")
    s = lax.axis_index("s")
    t0 = s * _TOK_PER_TILE
    for e in range(NUM_EXPERTS):
        pltpu.sync_copy(pos.at[e, pl.ds(t0, _TOK_PER_TILE)], pos_v.at[e])

    def one_chunk(y, k):
        copies = [
            pltpu.async_copy(
                y.at[pos_v.at[e, pl.ds(k * _C_SUB, _C_SUB)]],
                gbuf.at[e], sem)
            for e in range(NUM_EXPERTS)
        ]
        for cp in copies:
            cp.wait()

        def row_body(r, _):
            def col_body(j, _):
                sl = pl.ds(j * 16, 16)
                v = gbuf[0, r, sl]
                for e in range(1, NUM_EXPERTS):
                    v = v + gbuf[e, r, sl]
                gbuf[0, r, sl] = v
                return 0

            lax.fori_loop(0, HALF // 16, col_body, 0)
            return 0

        lax.fori_loop(0, _C_SUB, row_body, 0)
        pltpu.sync_copy(gbuf.at[0],
                        out.at[pl.ds(t0 + k * _C_SUB, _C_SUB), c])

    @pl.when(c == 0)
    def _():
        for k in range(_TOK_PER_TILE // _C_SUB):
            one_chunk(yl, k)

    @pl.when(c == 1)
    def _():
        for k in range(_TOK_PER_TILE // _C_SUB):
            one_chunk(yr, k)


def _combine(yl, yr, pos):
    return pl.kernel(
        _combine_body,
        out_type=jax.ShapeDtypeStruct((T, 2, HALF), jnp.float32),
        mesh=plsc.VectorSubcoreMesh(**_MESH),
        scratch_types=(
            pltpu.VMEM((NUM_EXPERTS, _TOK_PER_TILE), jnp.int32),
            pltpu.VMEM((NUM_EXPERTS, _C_SUB, HALF), jnp.float32),
            pltpu.SemaphoreType.DMA,
        ),
    )(yl, yr, pos)


def kernel(hidden_states, expert_indices, gate_up_w, down_w):
    maskT = expert_indices.T
    tok, cnt16, pos = _route(maskT)
    counts = jnp.concatenate(
        [cnt16[:, 0], jnp.zeros((8,), jnp.int32)])
    xg = _gather(hidden_states, tok)
    yl, yr = _gmm(counts, xg, gate_up_w, down_w)
    out = _combine(yl.reshape(Y_ROWS, HALF), yr.reshape(Y_ROWS, HALF), pos)
    return out.reshape(T, D_MODEL)


# speedup vs baseline: 1.6341x; 1.6341x over previous
import functools

import jax
import jax.numpy as jnp
from jax import lax
from jax.experimental import pallas as pl
from jax.experimental.pallas import tpu as pltpu
from jax.experimental.pallas import tpu_sc as plsc

NUM_EXPERTS = 8
D_MODEL = 1024
D_FF = 4096
T = 2048

S_TOT = NUM_EXPERTS * T
Y_ROWS = (NUM_EXPERTS + 1) * T
HALF = D_MODEL // 2
FF_CHUNK = 512
N_FF = D_FF // FF_CHUNK
ROW_SUB = 256

_MESH = dict(core_axis_name="c", subcore_axis_name="s")


def _route_body(maskT, tok, cnt, pos, mask_v, tok_v, pos_v, cnt_v):
    c = lax.axis_index("c")
    s = lax.axis_index("s")
    w = s * 2 + c

    @pl.when(w < NUM_EXPERTS)
    def _():
        pltpu.sync_copy(maskT.at[w], mask_v)
        for k in range(len(tok_v) // 16):
            tok_v[pl.ds(k * 16, 16)] = jnp.zeros((16,), jnp.int32)
        lane = lax.broadcasted_iota(jnp.int32, (16,), 0)
        group_base = w * T

        def chunk(k, ptr):
            mv = mask_v[pl.ds(k * 16, 16)]
            ids = lane + k * 16
            exc = plsc.cumsum(mv) - mv
            dst = jnp.where(mv > 0, ptr + exc, T + lane)
            plsc.store_scatter(tok_v, [dst], ids)
            pos_v[pl.ds(k * 16, 16)] = jnp.where(
                mv > 0, group_base + ptr + exc, S_TOT + ids)
            return ptr + jnp.sum(mv)

        ptr = lax.fori_loop(0, T // 16, chunk, jnp.int32(0))
        pltpu.sync_copy(tok_v.at[pl.ds(0, T)], tok.at[w])
        pltpu.sync_copy(pos_v, pos.at[w])
        cnt_v[...] = jnp.full((16,), ptr, jnp.int32)
        pltpu.sync_copy(cnt_v, cnt.at[w])


def _route(maskT):
    return pl.kernel(
        _route_body,
        out_type=(
            jax.ShapeDtypeStruct((NUM_EXPERTS, T), jnp.int32),
            jax.ShapeDtypeStruct((NUM_EXPERTS, 16), jnp.int32),
            jax.ShapeDtypeStruct((NUM_EXPERTS, T), jnp.int32),
        ),
        mesh=plsc.VectorSubcoreMesh(**_MESH),
        scratch_types=(
            pltpu.VMEM((T,), jnp.int32),
            pltpu.VMEM((T + 16, ), jnp.int32),
            pltpu.VMEM((T,), jnp.int32),
            pltpu.VMEM((16,), jnp.int32),
        ),
        compiler_params=pltpu.CompilerParams(needs_layout_passes=False),
    )(maskT)


_G_CHUNK = 64


def _gather_body(x, tok, cnt, xg, idx_v, xbuf, cnt_v, sem):
    c = lax.axis_index("c")
    s = lax.axis_index("s")
    w = s * 2 + c
    e = w // 4
    lo = (w % 4) * 512
    pltpu.sync_copy(cnt.at[e], cnt_v)
    cv = cnt_v[...]
    mine = jnp.clip(cv[0] - lo, 0, 512)
    nch = (mine + _G_CHUNK - 1) // _G_CHUNK
    for j in range(512 // _G_CHUNK):
        pltpu.sync_copy(tok.at[e, pl.ds(lo + j * _G_CHUNK, _G_CHUNK)],
                        idx_v.at[j])

    def chunk(k, _):
        pltpu.async_copy(x.at[idx_v.at[k]], xbuf, sem).wait()
        pltpu.sync_copy(xbuf, xg.at[pl.ds(w * 512 + k * _G_CHUNK, _G_CHUNK)])
        return 0

    lax.fori_loop(0, nch, chunk, 0)


def _gather(x, tok, cnt):
    return pl.kernel(
        _gather_body,
        out_type=jax.ShapeDtypeStruct((S_TOT, D_MODEL), jnp.float32),
        mesh=plsc.VectorSubcoreMesh(**_MESH),
        scratch_types=(
            pltpu.VMEM((512 // _G_CHUNK, _G_CHUNK), jnp.int32),
            pltpu.VMEM((_G_CHUNK, D_MODEL), jnp.float32),
            pltpu.VMEM((16,), jnp.int32),
            pltpu.SemaphoreType.DMA,
        ),
    )(x, tok, cnt)


def _gmm_body(cnt_ref, xg_ref, wg_ref, wu_ref, wd_ref, yl_ref, yr_ref):
    e = pl.program_id(0)
    f = pl.program_id(1)
    cnt = cnt_ref[e]
    nblk = (cnt + ROW_SUB - 1) // ROW_SUB

    @pl.when((e == NUM_EXPERTS) & (f == 0))
    def _():
        yl_ref[...] = jnp.zeros_like(yl_ref)
        yr_ref[...] = jnp.zeros_like(yr_ref)

    def sub(j, _):
        rows = pl.ds(j * ROW_SUB, ROW_SUB)
        xs = xg_ref[0, rows, :]
        g = jnp.dot(xs, wg_ref[0], preferred_element_type=jnp.float32)
        u = jnp.dot(xs, wu_ref[0], preferred_element_type=jnp.float32)
        h = u * (g * jax.nn.sigmoid(g))
        part = jnp.dot(h, wd_ref[0], preferred_element_type=jnp.float32)

        @pl.when(f == 0)
        def _():
            yl_ref[0, rows, :] = part[:, :HALF]
            yr_ref[0, rows, :] = part[:, HALF:]

        @pl.when(f != 0)
        def _():
            yl_ref[0, rows, :] += part[:, :HALF]
            yr_ref[0, rows, :] += part[:, HALF:]

        return 0

    lax.fori_loop(0, nblk, sub, 0)


def _gmm(counts, xg, gate_up_w, down_w):
    ce = lambda e: jnp.minimum(e, NUM_EXPERTS - 1)
    grid_spec = pltpu.PrefetchScalarGridSpec(
        num_scalar_prefetch=1,
        grid=(NUM_EXPERTS + 1, N_FF),
        in_specs=[
            pl.BlockSpec((1, T, D_MODEL), lambda e, f, cnt: (ce(e), 0, 0)),
            pl.BlockSpec((1, D_MODEL, FF_CHUNK),
                         lambda e, f, cnt: (ce(e), 0, f)),
            pl.BlockSpec((1, D_MODEL, FF_CHUNK),
                         lambda e, f, cnt: (ce(e), 0, N_FF + f)),
            pl.BlockSpec((1, FF_CHUNK, D_MODEL),
                         lambda e, f, cnt: (ce(e), f, 0)),
        ],
        out_specs=[
            pl.BlockSpec((1, T, HALF), lambda e, f, cnt: (e, 0, 0)),
            pl.BlockSpec((1, T, HALF), lambda e, f, cnt: (e, 0, 0)),
        ],
    )
    return pl.pallas_call(
        _gmm_body,
        grid_spec=grid_spec,
        out_shape=(
            jax.ShapeDtypeStruct((NUM_EXPERTS + 1, T, HALF), jnp.float32),
            jax.ShapeDtypeStruct((NUM_EXPERTS + 1, T, HALF), jnp.float32),
        ),
    )(counts, xg.reshape(NUM_EXPERTS, T, D_MODEL), gate_up_w, gate_up_w,
      down_w)


_TOK_PER_TILE = T // 16
_C_SUB = 16


def _combine_body(yl, yr, pos, out, pos_v, gbuf, sem):
    c = lax.axis_index("c")
    s = lax.axis_index("s")
    t0 = s * _TOK_PER_TILE
    for e in range(NUM_EXPERTS):
        pltpu.sync_copy(pos.at[e, pl.ds(t0, _TOK_PER_TILE)], pos_v.at[e])

    def one_chunk(y, k):
        copies = [
            pltpu.async_copy(
                y.at[pos_v.at[e, pl.ds(k * _C_SUB, _C_SUB)]],
                gbuf.at[e], sem)
            for e in range(NUM_EXPERTS)
        ]
        for cp in copies:
            cp.wait()

        def row_body(r, _):
            def col_body(j, _):
                sl = pl.ds(j * 16, 16)
                v = gbuf[0, r, sl]
                for e in range(1, NUM_EXPERTS):
                    v = v + gbuf[e, r, sl]
                gbuf[0, r, sl] = v
                return 0

            lax.fori_loop(0, HALF // 16, col_body, 0)
            return 0

        lax.fori_loop(0, _C_SUB, row_body, 0)
        pltpu.sync_copy(gbuf.at[0],
                        out.at[pl.ds(t0 + k * _C_SUB, _C_SUB), c])

    @pl.when(c == 0)
    def _():
        for k in range(_TOK_PER_TILE // _C_SUB):
            one_chunk(yl, k)

    @pl.when(c == 1)
    def _():
        for k in range(_TOK_PER_TILE // _C_SUB):
            one_chunk(yr, k)


def _combine(yl, yr, pos):
    return pl.kernel(
        _combine_body,
        out_type=jax.ShapeDtypeStruct((T, 2, HALF), jnp.float32),
        mesh=plsc.VectorSubcoreMesh(**_MESH),
        scratch_types=(
            pltpu.VMEM((NUM_EXPERTS, _TOK_PER_TILE), jnp.int32),
            pltpu.VMEM((NUM_EXPERTS, _C_SUB, HALF), jnp.float32),
            pltpu.SemaphoreType.DMA,
        ),
    )(yl, yr, pos)


def kernel(hidden_states, expert_indices, gate_up_w, down_w):
    maskT = expert_indices.T
    tok, cnt16, pos = _route(maskT)
    counts = jnp.concatenate(
        [cnt16[:, 0], jnp.zeros((8,), jnp.int32)])
    xg = _gather(hidden_states, tok, cnt16)
    yl, yr = _gmm(counts, xg, gate_up_w, down_w)
    out = _combine(yl.reshape(Y_ROWS, HALF), yr.reshape(Y_ROWS, HALF), pos)
    return out.reshape(T, D_MODEL)
